# baseline (speedup 1.0000x reference)
import functools

import jax
import jax.numpy as jnp
from jax import lax
from jax.experimental import pallas as pl
from jax.experimental.pallas import tpu as pltpu
from jax.experimental.pallas import tpu_sc as plsc

_NC = 2
_NS = 16
_NW = _NC * _NS
_C = 128


def kernel(x, weight):
    batch_shape = x.shape
    emb = weight.shape[-1]
    n = x.size
    per_w = n // _NW
    chunks = per_w // _C
    flat_idx = x.reshape(n).astype(jnp.int32)

    mesh = plsc.VectorSubcoreMesh(core_axis_name="c", subcore_axis_name="s")

    @functools.partial(
        pl.kernel,
        mesh=mesh,
        compiler_params=pltpu.CompilerParams(use_tc_tiling_on_sc=False),
        out_type=jax.ShapeDtypeStruct((n, emb), weight.dtype),
        scratch_types=[
            pltpu.VMEM((_C,), jnp.int32),
            pltpu.VMEM((_C, emb), weight.dtype),
            pltpu.SemaphoreType.DMA,
        ],
    )
    def gather_kernel(table_hbm, idx_hbm, out_hbm, idx_v, rows_v, sem):
        wid = lax.axis_index("s") * _NC + lax.axis_index("c")
        base = wid * per_w

        @pl.loop(0, chunks)
        def _(i):
            off = base + i * _C
            pltpu.sync_copy(idx_hbm.at[pl.ds(off, _C)], idx_v)
            pltpu.async_copy(table_hbm.at[idx_v], rows_v, sem).wait()
            pltpu.sync_copy(rows_v, out_hbm.at[pl.ds(off, _C)])

    out = gather_kernel(weight, flat_idx)
    return out.reshape(batch_shape + (emb,))


# speedup vs baseline: 1.2138x; 1.2138x over previous
import dataclasses
import functools

import jax
import jax.numpy as jnp
from jax import lax
from jax.experimental import pallas as pl
from jax.experimental.pallas import tpu as pltpu
from jax.experimental.pallas import tpu_sc as plsc

_NC = 2
_NS = 16
_NW = _NC * _NS
_B = 128
_L = 16


def kernel(x, weight):
    nb, nj = x.shape
    nv, emb = weight.shape
    xT = x.T
    w4 = weight.reshape(nv // 4, 4 * emb)

    chunks_per_j = nb // _B
    total_chunks = nj * chunks_per_j
    per_w = total_chunks // _NW

    mesh = plsc.VectorSubcoreMesh(core_axis_name="c", subcore_axis_name="s")

    cp = pltpu.CompilerParams()
    if "needs_layout_passes" in pltpu.CompilerParams.__dataclass_fields__:
        cp = dataclasses.replace(cp, needs_layout_passes=False)

    @functools.partial(
        pl.kernel,
        mesh=mesh,
        compiler_params=cp,
        out_type=jax.ShapeDtypeStruct((nj, emb, nb), weight.dtype),
        scratch_types=[
            pltpu.VMEM((8, _B), jnp.int32),
            pltpu.VMEM((_B,), jnp.int32),
            pltpu.VMEM((_B,), jnp.int32),
            pltpu.VMEM((_B, 4 * emb), jnp.float32),
            pltpu.VMEM((1, emb, _B), jnp.float32),
            pltpu.SemaphoreType.DMA,
        ],
    )
    def gather_kernel(xT_hbm, w4_hbm, out_hbm, idx_v, g_v, off_v, gath_v,
                      out_v, sem):
        wid = lax.axis_index("s") * _NC + lax.axis_index("c")

        @pl.loop(0, per_w)
        def _(k):
            cid = wid * per_w + k
            j = cid >> 7
            b0 = (cid & (chunks_per_j - 1)) << 7
            j8 = pl.multiple_of((j >> 3) << 3, 8)
            jr = j & 7

            pltpu.sync_copy(
                xT_hbm.at[pl.ds(j8, 8), pl.ds(pl.multiple_of(b0, _B), _B)],
                idx_v)

            for s in range(_B // _L):
                raw = idx_v[jr, pl.ds(s * _L, _L)]
                g_v[pl.ds(s * _L, _L)] = raw >> 2
                off_v[pl.ds(s * _L, _L)] = (raw & 3) << 5

            pltpu.async_copy(w4_hbm.at[g_v], gath_v, sem).wait()

            for s in range(_B // _L):
                rows = jax.lax.iota(jnp.int32, _L) + s * _L
                cols0 = off_v[pl.ds(s * _L, _L)]
                for f in range(emb):
                    vals = plsc.load_gather(gath_v, [rows, cols0 + f])
                    out_v[0, f, pl.ds(s * _L, _L)] = vals

            pltpu.sync_copy(out_v,
                            out_hbm.at[pl.ds(j, 1), pl.ds(0, emb),
                                       pl.ds(pl.multiple_of(b0, _B), _B)])

    outT = gather_kernel(xT, w4)
    return outT.transpose(2, 0, 1)


# speedup vs baseline: 1.4543x; 1.1981x over previous
import dataclasses
import functools

import jax
import jax.numpy as jnp
from jax import lax
from jax.experimental import pallas as pl
from jax.experimental.pallas import tpu as pltpu
from jax.experimental.pallas import tpu_sc as plsc

_NC = 2
_NS = 16
_NW = _NC * _NS
_B = 128
_L = 16


def kernel(x, weight):
    nb, nj = x.shape
    nv, emb = weight.shape
    xT = x.T
    w4 = weight.reshape(nv // 4, 4 * emb)

    chunks_per_j = nb // _B
    total_chunks = nj * chunks_per_j
    per_w = total_chunks // _NW

    mesh = plsc.VectorSubcoreMesh(core_axis_name="c", subcore_axis_name="s")

    cp = pltpu.CompilerParams()
    if "needs_layout_passes" in pltpu.CompilerParams.__dataclass_fields__:
        cp = dataclasses.replace(cp, needs_layout_passes=False)

    @functools.partial(
        pl.kernel,
        mesh=mesh,
        compiler_params=cp,
        out_type=jax.ShapeDtypeStruct((nj, emb, nb), weight.dtype),
        scratch_types=[
            pltpu.VMEM((2, 8, _B), jnp.int32),
            pltpu.VMEM((2, _B), jnp.int32),
            pltpu.VMEM((2, _B), jnp.int32),
            pltpu.VMEM((2, _B, 4 * emb), jnp.float32),
            pltpu.VMEM((2, 1, emb, _B), jnp.float32),
            pltpu.SemaphoreType.DMA,
            pltpu.SemaphoreType.DMA,
            pltpu.SemaphoreType.DMA,
            pltpu.SemaphoreType.DMA,
            pltpu.SemaphoreType.DMA,
            pltpu.SemaphoreType.DMA,
        ],
    )
    def gather_kernel(xT_hbm, w4_hbm, out_hbm, idx_v, g_v, off_v, gath_v,
                      out_v, isem0, isem1, gsem0, gsem1, osem0, osem1):
        wid = lax.axis_index("s") * _NC + lax.axis_index("c")
        base = wid * per_w
        isem = (isem0, isem1)
        gsem = (gsem0, gsem1)
        osem = (osem0, osem1)

        def chunk_coords(i):
            cid = base + i
            j = cid >> 7
            b0 = pl.multiple_of((cid & (chunks_per_j - 1)) << 7, _B)
            j8 = pl.multiple_of((j >> 3) << 3, 8)
            return j, j8, b0

        def start_idx(i, p):
            _, j8, b0 = chunk_coords(i)
            pltpu.make_async_copy(
                xT_hbm.at[pl.ds(j8, 8), pl.ds(b0, _B)],
                idx_v.at[p], isem[p]).start()

        def compute_and_start_gather(i, p):
            j, _, _ = chunk_coords(i)
            jr = j & 7
            pltpu.make_async_copy(
                xT_hbm.at[pl.ds(0, 8), pl.ds(0, _B)],
                idx_v.at[p], isem[p]).wait()
            for s in range(_B // _L):
                raw = idx_v[p, jr, pl.ds(s * _L, _L)]
                g_v[p, pl.ds(s * _L, _L)] = raw >> 2
                off_v[p, pl.ds(s * _L, _L)] = (raw & 3) << 5
            pltpu.make_async_copy(
                w4_hbm.at[g_v.at[p]], gath_v.at[p], gsem[p]).start()

        def extract_and_write(i, p):
            j, _, b0 = chunk_coords(i)
            pltpu.make_async_copy(
                w4_hbm.at[g_v.at[p]], gath_v.at[p], gsem[p]).wait()
            for s in range(_B // _L):
                rows = jax.lax.iota(jnp.int32, _L) + s * _L
                cols0 = off_v[p, pl.ds(s * _L, _L)]
                for f in range(emb):
                    vals = plsc.load_gather(gath_v.at[p], [rows, cols0 + f])
                    out_v[p, 0, f, pl.ds(s * _L, _L)] = vals

            @pl.when(i >= 2)
            def _():
                pltpu.make_async_copy(
                    out_v.at[p],
                    out_hbm.at[pl.ds(0, 1), pl.ds(0, emb), pl.ds(0, _B)],
                    osem[p]).wait()

            pltpu.make_async_copy(
                out_v.at[p],
                out_hbm.at[pl.ds(j, 1), pl.ds(0, emb), pl.ds(b0, _B)],
                osem[p]).start()

        start_idx(0, 0)
        start_idx(1, 1)
        compute_and_start_gather(0, 0)

        @pl.loop(0, per_w // 2)
        def _(g):
            for half in range(2):
                i = g * 2 + half
                p = half
                q = 1 - half

                @pl.when(i + 1 < per_w)
                def _():
                    compute_and_start_gather(i + 1, q)

                extract_and_write(i, p)

                @pl.when(i + 2 < per_w)
                def _():
                    start_idx(i + 2, p)

        for p in range(2):
            pltpu.make_async_copy(
                out_v.at[p],
                out_hbm.at[pl.ds(0, 1), pl.ds(0, emb), pl.ds(0, _B)],
                osem[p]).wait()

    outT = gather_kernel(xT, w4)
    return outT.transpose(2, 0, 1)
